# baseline (speedup 1.0000x reference)
from functools import partial

import jax
import jax.numpy as jnp
from jax.experimental import pallas as pl
from jax.experimental.pallas import tpu as pltpu


def _ghost_fused_kernel(x_ref, pww_ref, pwb_ref, cw_ref, cb_ref,
                        dww_ref, dwb_ref, o_ref,
                        x1p_ref, x2p_ref, ev_ref, od_ref, sp_ref,
                        *, H, W, L, Ho, Wo):
    xr = x_ref[0]
    xs = xr.reshape(2 * xr.shape[1], H * W)
    wv = pww_ref[...]

    x1 = jax.lax.dot_general(xs, wv, (((0,), (0,)), ((), ())),
                             preferred_element_type=jnp.float32)
    x1 = jnp.maximum(x1 + pwb_ref[...], 0.0)
    x1 = x1.reshape(H, W, L)

    zrow = jnp.zeros((1, W + 2, L), jnp.float32)
    zcol = jnp.zeros((H + 2, 1, L), jnp.float32)
    for ref in (x1p_ref, x2p_ref):
        ref[0:1] = zrow
        ref[H + 1:H + 2] = zrow
        ref[:, 0:1] = zcol
        ref[:, W + 1:W + 2] = zcol

    x1p_ref[1:H + 1, 1:W + 1, :] = x1

    cwv = cw_ref[...]
    cbv = cb_ref[...].reshape(1, 1, L)
    TH = 8
    for t in range(H // TH):
        r0 = TH * t
        slab = x1p_ref[r0:r0 + TH + 2, :, :]
        acc = jnp.zeros((TH, W, L), jnp.float32)
        for ky in range(3):
            for kx in range(3):
                acc = acc + (slab[ky:ky + TH, kx:kx + W, :]
                             * cwv[ky, kx].reshape(1, 1, L))
        x2p_ref[r0 + 1:r0 + 1 + TH, 1:W + 1, :] = jnp.maximum(acc + cbv, 0.0)

    dwv = dww_ref[...]
    for half, src in ((0, x1p_ref), (1, x2p_ref)):
        ev_ref[:, 0:Wo + 1, :] = src[:, pl.ds(0, Wo + 1, stride=2), :]
        od_ref[:, 0:Wo + 1, :] = src[:, pl.ds(1, Wo + 1, stride=2), :]
        sacc = jnp.zeros((Ho, Wo, L), jnp.float32)
        for ky in range(3):
            for kx, (tref, off) in enumerate(
                    ((ev_ref, 0), (od_ref, 0), (ev_ref, 1))):
                taps = tref[pl.ds(ky, Ho, stride=2), off:off + Wo, :]
                sacc = sacc + taps * dwv[ky, kx, half].reshape(1, 1, L)
        out = sacc + dwb_ref[half].reshape(1, 1, L)
        sp_ref[:, 0:Wo, :] = out
        v = sp_ref[...].reshape(Ho * 128, L)
        t = jnp.transpose(v)
        t3 = t.reshape(L, Ho, 128)
        o_ref[0, :, half] = t3[:, :, 0:Wo].reshape(2, L // 2, Ho, Wo)


def kernel(x_nchw, pw_w, pw_scale, pw_bias, cheap_w, cheap_scale, cheap_bias,
           dw_w, dw_scale, dw_bias):
    N, Cin, H, W = x_nchw.shape
    C = pw_w.shape[1]
    L = 2 * C
    N2 = N // 2
    Ho = (H - 1) // 2 + 1
    Wo = (W - 1) // 2 + 1

    pw_eff = pw_w * pw_scale.reshape(1, C)
    cw_eff = cheap_w * cheap_scale.reshape(1, 1, C)
    dw_eff = dw_w * dw_scale.reshape(1, 1, 2 * C)

    z = jnp.zeros((Cin, C), jnp.float32)
    w2 = jnp.concatenate([jnp.concatenate([pw_eff, z], axis=1),
                          jnp.concatenate([z, pw_eff], axis=1)], axis=0)
    tile2 = lambda v: jnp.tile(v.reshape(1, -1), (1, 2))

    body = partial(_ghost_fused_kernel, H=H, W=W, L=L, Ho=Ho, Wo=Wo)
    out6 = pl.pallas_call(
        body,
        out_shape=jax.ShapeDtypeStruct((N2, 2, 2, C, Ho, Wo), jnp.float32),
        grid=(N2,),
        in_specs=[
            pl.BlockSpec((1, 2, Cin, H * W), lambda n: (n, 0, 0, 0)),
            pl.BlockSpec((2 * Cin, L), lambda n: (0, 0)),
            pl.BlockSpec((1, L), lambda n: (0, 0)),
            pl.BlockSpec((3, 3, L), lambda n: (0, 0, 0)),
            pl.BlockSpec((1, L), lambda n: (0, 0)),
            pl.BlockSpec((3, 3, 2, L), lambda n: (0, 0, 0, 0)),
            pl.BlockSpec((2, L), lambda n: (0, 0)),
        ],
        out_specs=pl.BlockSpec((1, 2, 2, C, Ho, Wo),
                               lambda n: (n, 0, 0, 0, 0, 0)),
        scratch_shapes=[
            pltpu.VMEM((H + 2, W + 2, L), jnp.float32),
            pltpu.VMEM((H + 2, W + 2, L), jnp.float32),
            pltpu.VMEM((H + 2, Wo + 4, L), jnp.float32),
            pltpu.VMEM((H + 2, Wo + 4, L), jnp.float32),
            pltpu.VMEM((Ho, 128, L), jnp.float32),
        ],
        compiler_params=pltpu.CompilerParams(
            dimension_semantics=("parallel",),
            vmem_limit_bytes=48 * 1024 * 1024),
    )(x_nchw.reshape(N2, 2, Cin, H * W), w2,
      tile2(pw_bias),
      jnp.tile(cw_eff, (1, 1, 2)), tile2(cheap_bias),
      jnp.tile(dw_eff.reshape(3, 3, 2, C), (1, 1, 1, 2)),
      jnp.tile(dw_bias.reshape(2, C), (1, 2)))

    return out6.reshape(N, 2 * C, Ho, Wo)


# speedup vs baseline: 1.1222x; 1.1222x over previous
from functools import partial

import jax
import jax.numpy as jnp
from jax.experimental import pallas as pl
from jax.experimental.pallas import tpu as pltpu


def _ghost_fused_kernel(x_ref, pww_ref, pwb_ref, cw_ref, cb_ref,
                        dww_ref, dwb_ref, o_ref,
                        x1p_ref, x2p_ref, sp_ref,
                        *, H, W, L, Ho, Wo):
    xr = x_ref[0]
    xs = xr.reshape(2 * xr.shape[1], H * W)
    wv = pww_ref[...]

    x1 = jax.lax.dot_general(xs, wv, (((0,), (0,)), ((), ())),
                             preferred_element_type=jnp.float32)
    x1 = jnp.maximum(x1 + pwb_ref[...], 0.0)
    x1 = x1.reshape(H, W, L)

    zrow = jnp.zeros((1, W + 2, L), jnp.float32)
    zcol = jnp.zeros((H + 2, 1, L), jnp.float32)
    for ref in (x1p_ref, x2p_ref):
        ref[0:1] = zrow
        ref[H + 1:H + 2] = zrow
        ref[:, 0:1] = zcol
        ref[:, W + 1:W + 2] = zcol

    x1p_ref[1:H + 1, 1:W + 1, :] = x1

    cwv = cw_ref[...]
    cbv = cb_ref[...].reshape(1, 1, L)
    acc = jnp.zeros((H, W, L), jnp.float32)
    for ky in range(3):
        for kx in range(3):
            acc = acc + (x1p_ref[ky:ky + H, kx:kx + W, :]
                         * cwv[ky, kx].reshape(1, 1, L))
    x2p_ref[1:H + 1, 1:W + 1, :] = jnp.maximum(acc + cbv, 0.0)

    dwv = dww_ref[...]
    for half, src in ((0, x1p_ref), (1, x2p_ref)):
        sacc = jnp.zeros((Ho, Wo, L), jnp.float32)
        for ky in range(3):
            for kx in range(3):
                taps = src[pl.ds(ky, Ho, stride=2),
                           pl.ds(kx, Wo, stride=2), :]
                sacc = sacc + taps * dwv[ky, kx, half].reshape(1, 1, L)
        out = sacc + dwb_ref[half].reshape(1, 1, L)
        sp_ref[:, 0:Wo, :] = out
        v = sp_ref[...].reshape(Ho * 128, L)
        t = jnp.transpose(v)
        t3 = t.reshape(L, Ho, 128)
        o_ref[0, :, half] = t3[:, :, 0:Wo].reshape(2, L // 2, Ho, Wo)


def kernel(x_nchw, pw_w, pw_scale, pw_bias, cheap_w, cheap_scale, cheap_bias,
           dw_w, dw_scale, dw_bias):
    N, Cin, H, W = x_nchw.shape
    C = pw_w.shape[1]
    L = 2 * C
    N2 = N // 2
    Ho = (H - 1) // 2 + 1
    Wo = (W - 1) // 2 + 1

    pw_eff = pw_w * pw_scale.reshape(1, C)
    cw_eff = cheap_w * cheap_scale.reshape(1, 1, C)
    dw_eff = dw_w * dw_scale.reshape(1, 1, 2 * C)

    z = jnp.zeros((Cin, C), jnp.float32)
    w2 = jnp.concatenate([jnp.concatenate([pw_eff, z], axis=1),
                          jnp.concatenate([z, pw_eff], axis=1)], axis=0)
    tile2 = lambda v: jnp.tile(v.reshape(1, -1), (1, 2))

    body = partial(_ghost_fused_kernel, H=H, W=W, L=L, Ho=Ho, Wo=Wo)
    out6 = pl.pallas_call(
        body,
        out_shape=jax.ShapeDtypeStruct((N2, 2, 2, C, Ho, Wo), jnp.float32),
        grid=(N2,),
        in_specs=[
            pl.BlockSpec((1, 2, Cin, H * W), lambda n: (n, 0, 0, 0)),
            pl.BlockSpec((2 * Cin, L), lambda n: (0, 0)),
            pl.BlockSpec((1, L), lambda n: (0, 0)),
            pl.BlockSpec((3, 3, L), lambda n: (0, 0, 0)),
            pl.BlockSpec((1, L), lambda n: (0, 0)),
            pl.BlockSpec((3, 3, 2, L), lambda n: (0, 0, 0, 0)),
            pl.BlockSpec((2, L), lambda n: (0, 0)),
        ],
        out_specs=pl.BlockSpec((1, 2, 2, C, Ho, Wo),
                               lambda n: (n, 0, 0, 0, 0, 0)),
        scratch_shapes=[
            pltpu.VMEM((H + 2, W + 2, L), jnp.float32),
            pltpu.VMEM((H + 2, W + 2, L), jnp.float32),
            pltpu.VMEM((Ho, 128, L), jnp.float32),
        ],
        compiler_params=pltpu.CompilerParams(
            dimension_semantics=("parallel",),
            vmem_limit_bytes=48 * 1024 * 1024),
    )(x_nchw.reshape(N2, 2, Cin, H * W), w2,
      tile2(pw_bias),
      jnp.tile(cw_eff, (1, 1, 2)), tile2(cheap_bias),
      jnp.tile(dw_eff.reshape(3, 3, 2, C), (1, 1, 1, 2)),
      jnp.tile(dw_bias.reshape(2, C), (1, 2)))

    return out6.reshape(N, 2 * C, Ho, Wo)


# speedup vs baseline: 1.1233x; 1.0010x over previous
from functools import partial

import jax
import jax.numpy as jnp
from jax.experimental import pallas as pl
from jax.experimental.pallas import tpu as pltpu


def _ghost_fused_kernel(x_ref, pww_ref, pwb_ref, cw_ref, cb_ref,
                        dww_ref, dwb_ref, o_ref,
                        x1p_ref, x2p_ref, sp_ref,
                        *, H, W, L, Ho, Wo):
    xr = x_ref[0]
    xs = xr.reshape(2 * xr.shape[1], H * W)
    wv = pww_ref[...]

    x1 = jax.lax.dot_general(xs, wv, (((0,), (0,)), ((), ())),
                             preferred_element_type=jnp.float32)
    x1 = jnp.maximum(x1 + pwb_ref[...], 0.0)
    x1 = x1.reshape(H, W, L)

    zrow = jnp.zeros((1, W + 2, L), jnp.float32)
    zcol = jnp.zeros((H + 2, 1, L), jnp.float32)
    for ref in (x1p_ref, x2p_ref):
        ref[0:1] = zrow
        ref[H + 1:H + 2] = zrow
        ref[:, 0:1] = zcol
        ref[:, W + 1:W + 2] = zcol

    x1p_ref[1:H + 1, 1:W + 1, :] = x1

    cwv = cw_ref[...]
    cbv = cb_ref[...].reshape(1, 1, L)
    TB = 8
    for t in range(H // TB):
        r0 = TB * t
        acc = jnp.zeros((TB, W, L), jnp.float32)
        for ky in range(3):
            for kx in range(3):
                acc = acc + (x1p_ref[r0 + ky:r0 + ky + TB, kx:kx + W, :]
                             * cwv[ky, kx].reshape(1, 1, L))
        x2p_ref[r0 + 1:r0 + 1 + TB, 1:W + 1, :] = jnp.maximum(acc + cbv, 0.0)

    dwv = dww_ref[...]
    HB = Ho // 2
    for half, src in ((0, x1p_ref), (1, x2p_ref)):
        for b in range(2):
            h0 = HB * b
            sacc = jnp.zeros((HB, Wo, L), jnp.float32)
            for ky in range(3):
                for kx in range(3):
                    taps = src[pl.ds(2 * h0 + ky, HB, stride=2),
                               pl.ds(kx, Wo, stride=2), :]
                    sacc = sacc + taps * dwv[ky, kx, half].reshape(1, 1, L)
            sp_ref[h0:h0 + HB, 0:Wo, :] = sacc + dwb_ref[half].reshape(1, 1, L)
        v = sp_ref[...].reshape(Ho * 128, L)
        t = jnp.transpose(v)
        t3 = t.reshape(L, Ho, 128)
        o_ref[0, :, half] = t3[:, :, 0:Wo].reshape(2, L // 2, Ho, Wo)


def kernel(x_nchw, pw_w, pw_scale, pw_bias, cheap_w, cheap_scale, cheap_bias,
           dw_w, dw_scale, dw_bias):
    N, Cin, H, W = x_nchw.shape
    C = pw_w.shape[1]
    L = 2 * C
    N2 = N // 2
    Ho = (H - 1) // 2 + 1
    Wo = (W - 1) // 2 + 1

    pw_eff = pw_w * pw_scale.reshape(1, C)
    cw_eff = cheap_w * cheap_scale.reshape(1, 1, C)
    dw_eff = dw_w * dw_scale.reshape(1, 1, 2 * C)

    z = jnp.zeros((Cin, C), jnp.float32)
    w2 = jnp.concatenate([jnp.concatenate([pw_eff, z], axis=1),
                          jnp.concatenate([z, pw_eff], axis=1)], axis=0)
    tile2 = lambda v: jnp.tile(v.reshape(1, -1), (1, 2))

    body = partial(_ghost_fused_kernel, H=H, W=W, L=L, Ho=Ho, Wo=Wo)
    out6 = pl.pallas_call(
        body,
        out_shape=jax.ShapeDtypeStruct((N2, 2, 2, C, Ho, Wo), jnp.float32),
        grid=(N2,),
        in_specs=[
            pl.BlockSpec((1, 2, Cin, H * W), lambda n: (n, 0, 0, 0)),
            pl.BlockSpec((2 * Cin, L), lambda n: (0, 0)),
            pl.BlockSpec((1, L), lambda n: (0, 0)),
            pl.BlockSpec((3, 3, L), lambda n: (0, 0, 0)),
            pl.BlockSpec((1, L), lambda n: (0, 0)),
            pl.BlockSpec((3, 3, 2, L), lambda n: (0, 0, 0, 0)),
            pl.BlockSpec((2, L), lambda n: (0, 0)),
        ],
        out_specs=pl.BlockSpec((1, 2, 2, C, Ho, Wo),
                               lambda n: (n, 0, 0, 0, 0, 0)),
        scratch_shapes=[
            pltpu.VMEM((H + 2, W + 2, L), jnp.float32),
            pltpu.VMEM((H + 2, W + 2, L), jnp.float32),
            pltpu.VMEM((Ho, 128, L), jnp.float32),
        ],
        compiler_params=pltpu.CompilerParams(
            dimension_semantics=("parallel",),
            vmem_limit_bytes=48 * 1024 * 1024),
    )(x_nchw.reshape(N2, 2, Cin, H * W), w2,
      tile2(pw_bias),
      jnp.tile(cw_eff, (1, 1, 2)), tile2(cheap_bias),
      jnp.tile(dw_eff.reshape(3, 3, 2, C), (1, 1, 1, 2)),
      jnp.tile(dw_bias.reshape(2, C), (1, 2)))

    return out6.reshape(N, 2 * C, Ho, Wo)
